# baseline (speedup 1.0000x reference)
import functools

import jax
import jax.numpy as jnp
from jax import lax
from jax.experimental import pallas as pl
from jax.experimental.pallas import tpu as pltpu
from jax.experimental.pallas import tpu_sc as plsc


def _sc_gather(ids_flat, word_table, n_tokens, d):
    info = plsc.get_sparse_core_info()
    nc, ns = info.num_cores, info.num_subcores
    nw = nc * ns
    chunk = 64
    tok_per_w = n_tokens // nw
    n_pairs = tok_per_w // (2 * chunk)
    assert tok_per_w * nw == n_tokens and n_pairs * 2 * chunk == tok_per_w

    mesh = plsc.VectorSubcoreMesh(core_axis_name="c", subcore_axis_name="s")

    @functools.partial(
        pl.kernel,
        mesh=mesh,
        out_type=jax.ShapeDtypeStruct((n_tokens, d), jnp.float32),
        scratch_types=[
            pltpu.VMEM((chunk,), jnp.int32),
            pltpu.VMEM((chunk,), jnp.int32),
            pltpu.VMEM((chunk, d), jnp.float32),
            pltpu.VMEM((chunk, d), jnp.float32),
            pltpu.SemaphoreType.DMA,
            pltpu.SemaphoreType.DMA,
            pltpu.SemaphoreType.DMA,
            pltpu.SemaphoreType.DMA,
        ],
    )
    def gather_kernel(ids_hbm, table_hbm, out_hbm, idx_a, idx_b, rows_a,
                      rows_b, sem_ga, sem_gb, sem_wa, sem_wb):
        wid = lax.axis_index("s") * nc + lax.axis_index("c")
        base0 = wid * tok_per_w

        def out_at(tok):
            return out_hbm.at[pl.ds(tok, chunk)]

        pltpu.sync_copy(ids_hbm.at[pl.ds(base0, chunk)], idx_a)
        pltpu.async_copy(table_hbm.at[idx_a], rows_a, sem_ga)

        def body(g, carry):
            tok_a = base0 + (2 * g) * chunk
            tok_b = tok_a + chunk
            pltpu.make_async_copy(table_hbm.at[idx_a], rows_a, sem_ga).wait()

            @pl.when(g > 0)
            def _():
                pltpu.make_async_copy(rows_b, out_at(tok_b - 2 * chunk),
                                      sem_wb).wait()

            wr_a = pltpu.async_copy(rows_a, out_at(tok_a), sem_wa)
            pltpu.sync_copy(ids_hbm.at[pl.ds(tok_b, chunk)], idx_b)
            pltpu.async_copy(table_hbm.at[idx_b], rows_b, sem_gb)
            pltpu.make_async_copy(table_hbm.at[idx_b], rows_b, sem_gb).wait()
            wr_a.wait()
            pltpu.async_copy(rows_b, out_at(tok_b), sem_wb)

            @pl.when(g < n_pairs - 1)
            def _():
                tok_n = tok_b + chunk
                pltpu.sync_copy(ids_hbm.at[pl.ds(tok_n, chunk)], idx_a)
                pltpu.async_copy(table_hbm.at[idx_a], rows_a, sem_ga)

            return carry

        lax.fori_loop(0, n_pairs, body, 0)
        pltpu.make_async_copy(
            rows_b, out_at(base0 + tok_per_w - chunk), sem_wb).wait()

    return gather_kernel(ids_flat, word_table)


def _tc_epilogue(word_rows, type_ids, type_table, pos_emb, gamma, beta,
                 b_total, b_base, out_prev):
    b, s, d = word_rows.shape
    t = type_table.shape[0]

    def body(*refs):
        ids_ref, x_ref, ttab_ref, pos_ref, gamma_ref, beta_ref = refs[:6]
        out_ref, buf, sem = refs[-3], refs[-2], refs[-1]
        i = pl.program_id(0)
        slot = lax.rem(i, 2)

        def wr_copy(step, s_slot):
            return pltpu.make_async_copy(
                buf.at[s_slot], out_ref.at[pl.ds(b_base + step, 1)],
                sem.at[s_slot])

        @pl.when(i >= 2)
        def _():
            wr_copy(i - 2, slot).wait()

        x = x_ref[0]
        ids = ids_ref[0, 0]
        iota_t = lax.broadcasted_iota(jnp.int32, (1, t), 1)
        onehot = (ids[:, None] == iota_t).astype(jnp.float32)
        type_rows = jnp.dot(onehot, ttab_ref[...],
                            preferred_element_type=jnp.float32)
        y = x + type_rows + pos_ref[...]
        mean = jnp.mean(y, axis=-1, keepdims=True)
        c = y - mean
        var = jnp.mean(c * c, axis=-1, keepdims=True)
        norm = c * lax.rsqrt(var + 1e-12)
        res = (norm * gamma_ref[...] + beta_ref[...])[None]

        @pl.when(slot == 0)
        def _():
            buf[0] = res

        @pl.when(slot == 1)
        def _():
            buf[1] = res

        wr_copy(i, slot).start()

        @pl.when(i == b - 1)
        def _():
            @pl.when(jnp.int32(b) >= 2)
            def _():
                wr_copy(i - 1, 1 - slot).wait()
            wr_copy(i, slot).wait()

    in_specs = [
        pl.BlockSpec((1, 1, s), lambda i: (i, 0, 0)),
        pl.BlockSpec((1, s, d), lambda i: (i, 0, 0)),
        pl.BlockSpec((t, d), lambda i: (0, 0)),
        pl.BlockSpec((s, d), lambda i: (0, 0)),
        pl.BlockSpec((1, d), lambda i: (0, 0)),
        pl.BlockSpec((1, d), lambda i: (0, 0)),
    ]
    args = [type_ids.reshape(b, 1, s).astype(jnp.int32), word_rows,
            type_table, pos_emb, gamma.reshape(1, d), beta.reshape(1, d)]
    aliases = {}
    if out_prev is not None:
        in_specs.append(pl.BlockSpec(memory_space=pl.ANY))
        args.append(out_prev)
        aliases = {6: 0}
    return pl.pallas_call(
        body,
        grid=(b,),
        in_specs=in_specs,
        out_specs=pl.BlockSpec(memory_space=pl.ANY),
        out_shape=jax.ShapeDtypeStruct((b_total, s, d), jnp.float32),
        input_output_aliases=aliases,
        scratch_shapes=[
            pltpu.VMEM((2, 1, s, d), jnp.float32),
            pltpu.SemaphoreType.DMA((2,)),
        ],
    )(*args)


def kernel(input_word_ids, input_type_ids, word_table, type_table, pos_emb,
           gamma, beta):
    b, s = input_word_ids.shape
    v, d = word_table.shape
    n_slices = 16
    bs = b // n_slices
    assert bs * n_slices == b
    ids32 = input_word_ids.astype(jnp.int32)
    type32 = input_type_ids.astype(jnp.int32)
    rows = [
        _sc_gather(ids32[k * bs:(k + 1) * bs].reshape(-1), word_table,
                   bs * s, d).reshape(bs, s, d)
        for k in range(n_slices)
    ]
    out = None
    for k in range(n_slices):
        out = _tc_epilogue(rows[k], type32[k * bs:(k + 1) * bs], type_table,
                           pos_emb, gamma, beta, b, k * bs, out)
    return out


# speedup vs baseline: 1.0721x; 1.0721x over previous
import functools

import jax
import jax.numpy as jnp
from jax import lax
from jax.experimental import pallas as pl
from jax.experimental.pallas import tpu as pltpu
from jax.experimental.pallas import tpu_sc as plsc


def _sc_gather(ids_flat, word_table, n_tokens, d):
    info = plsc.get_sparse_core_info()
    nc, ns = info.num_cores, info.num_subcores
    nw = nc * ns
    chunk = 64
    tok_per_w = n_tokens // nw
    n_pairs = tok_per_w // (2 * chunk)
    assert tok_per_w * nw == n_tokens and n_pairs * 2 * chunk == tok_per_w

    mesh = plsc.VectorSubcoreMesh(core_axis_name="c", subcore_axis_name="s")

    @functools.partial(
        pl.kernel,
        mesh=mesh,
        out_type=jax.ShapeDtypeStruct((n_tokens, d), jnp.float32),
        scratch_types=[
            pltpu.VMEM((chunk,), jnp.int32),
            pltpu.VMEM((chunk,), jnp.int32),
            pltpu.VMEM((chunk, d), jnp.float32),
            pltpu.VMEM((chunk, d), jnp.float32),
            pltpu.SemaphoreType.DMA,
            pltpu.SemaphoreType.DMA,
            pltpu.SemaphoreType.DMA,
            pltpu.SemaphoreType.DMA,
        ],
    )
    def gather_kernel(ids_hbm, table_hbm, out_hbm, idx_a, idx_b, rows_a,
                      rows_b, sem_ga, sem_gb, sem_wa, sem_wb):
        wid = lax.axis_index("s") * nc + lax.axis_index("c")
        base0 = wid * tok_per_w

        def out_at(tok):
            return out_hbm.at[pl.ds(tok, chunk)]

        pltpu.sync_copy(ids_hbm.at[pl.ds(base0, chunk)], idx_a)
        pltpu.async_copy(table_hbm.at[idx_a], rows_a, sem_ga)

        def body(g, carry):
            tok_a = base0 + (2 * g) * chunk
            tok_b = tok_a + chunk
            pltpu.make_async_copy(table_hbm.at[idx_a], rows_a, sem_ga).wait()

            @pl.when(g > 0)
            def _():
                pltpu.make_async_copy(rows_b, out_at(tok_b - 2 * chunk),
                                      sem_wb).wait()

            wr_a = pltpu.async_copy(rows_a, out_at(tok_a), sem_wa)
            pltpu.sync_copy(ids_hbm.at[pl.ds(tok_b, chunk)], idx_b)
            pltpu.async_copy(table_hbm.at[idx_b], rows_b, sem_gb)
            pltpu.make_async_copy(table_hbm.at[idx_b], rows_b, sem_gb).wait()
            wr_a.wait()
            pltpu.async_copy(rows_b, out_at(tok_b), sem_wb)

            @pl.when(g < n_pairs - 1)
            def _():
                tok_n = tok_b + chunk
                pltpu.sync_copy(ids_hbm.at[pl.ds(tok_n, chunk)], idx_a)
                pltpu.async_copy(table_hbm.at[idx_a], rows_a, sem_ga)

            return carry

        lax.fori_loop(0, n_pairs, body, 0)
        pltpu.make_async_copy(
            rows_b, out_at(base0 + tok_per_w - chunk), sem_wb).wait()

    return gather_kernel(ids_flat, word_table)


def _tc_epilogue(word_rows, type_ids, type_table, pos_emb, gamma, beta,
                 b_total, b_base, out_prev):
    b, s, d = word_rows.shape
    t = type_table.shape[0]

    def body(*refs):
        ids_ref, x_ref, ttab_ref, pos_ref, gamma_ref, beta_ref = refs[:6]
        out_ref, buf, sem = refs[-3], refs[-2], refs[-1]
        i = pl.program_id(0)
        slot = lax.rem(i, 2)

        def wr_copy(step, s_slot):
            return pltpu.make_async_copy(
                buf.at[s_slot], out_ref.at[pl.ds(b_base + step, 1)],
                sem.at[s_slot])

        @pl.when(i >= 2)
        def _():
            wr_copy(i - 2, slot).wait()

        x = x_ref[0]
        ids = ids_ref[0, 0]
        iota_t = lax.broadcasted_iota(jnp.int32, (1, t), 1)
        onehot = (ids[:, None] == iota_t).astype(jnp.float32)
        type_rows = jnp.dot(onehot, ttab_ref[...],
                            preferred_element_type=jnp.float32)
        y = x + type_rows + pos_ref[...]
        mean = jnp.mean(y, axis=-1, keepdims=True)
        c = y - mean
        var = jnp.mean(c * c, axis=-1, keepdims=True)
        norm = c * lax.rsqrt(var + 1e-12)
        res = (norm * gamma_ref[...] + beta_ref[...])[None]

        @pl.when(slot == 0)
        def _():
            buf[0] = res

        @pl.when(slot == 1)
        def _():
            buf[1] = res

        wr_copy(i, slot).start()

        @pl.when(i == b - 1)
        def _():
            @pl.when(jnp.int32(b) >= 2)
            def _():
                wr_copy(i - 1, 1 - slot).wait()
            wr_copy(i, slot).wait()

    in_specs = [
        pl.BlockSpec((1, 1, s), lambda i: (i, 0, 0)),
        pl.BlockSpec((1, s, d), lambda i: (i, 0, 0)),
        pl.BlockSpec((t, d), lambda i: (0, 0)),
        pl.BlockSpec((s, d), lambda i: (0, 0)),
        pl.BlockSpec((1, d), lambda i: (0, 0)),
        pl.BlockSpec((1, d), lambda i: (0, 0)),
    ]
    args = [type_ids.reshape(b, 1, s).astype(jnp.int32), word_rows,
            type_table, pos_emb, gamma.reshape(1, d), beta.reshape(1, d)]
    aliases = {}
    if out_prev is not None:
        in_specs.append(pl.BlockSpec(memory_space=pl.ANY))
        args.append(out_prev)
        aliases = {6: 0}
    return pl.pallas_call(
        body,
        grid=(b,),
        in_specs=in_specs,
        out_specs=pl.BlockSpec(memory_space=pl.ANY),
        out_shape=jax.ShapeDtypeStruct((b_total, s, d), jnp.float32),
        input_output_aliases=aliases,
        scratch_shapes=[
            pltpu.VMEM((2, 1, s, d), jnp.float32),
            pltpu.SemaphoreType.DMA((2,)),
        ],
    )(*args)


def kernel(input_word_ids, input_type_ids, word_table, type_table, pos_emb,
           gamma, beta):
    b, s = input_word_ids.shape
    v, d = word_table.shape
    sizes = [8, 16, 16, 16, 16, 16, 16, 16, 8]
    assert sum(sizes) == b
    ids32 = input_word_ids.astype(jnp.int32)
    type32 = input_type_ids.astype(jnp.int32)
    starts = [sum(sizes[:k]) for k in range(len(sizes))]
    rows = [
        _sc_gather(ids32[b0:b0 + bs].reshape(-1), word_table,
                   bs * s, d).reshape(bs, s, d)
        for b0, bs in zip(starts, sizes)
    ]
    out = None
    for k, (b0, bs) in enumerate(zip(starts, sizes)):
        out = _tc_epilogue(rows[k], type32[b0:b0 + bs], type_table,
                           pos_emb, gamma, beta, b, b0, out)
    return out


# speedup vs baseline: 1.0861x; 1.0130x over previous
import functools

import jax
import jax.numpy as jnp
from jax import lax
from jax.experimental import pallas as pl
from jax.experimental.pallas import tpu as pltpu
from jax.experimental.pallas import tpu_sc as plsc


def _sc_gather(ids_flat, word_table, n_tokens, d):
    info = plsc.get_sparse_core_info()
    nc, ns = info.num_cores, info.num_subcores
    nw = nc * ns
    chunk = 64
    tok_per_w = n_tokens // nw
    n_pairs = tok_per_w // (2 * chunk)
    assert tok_per_w * nw == n_tokens and n_pairs * 2 * chunk == tok_per_w

    mesh = plsc.VectorSubcoreMesh(core_axis_name="c", subcore_axis_name="s")

    @functools.partial(
        pl.kernel,
        mesh=mesh,
        out_type=jax.ShapeDtypeStruct((n_tokens, d), jnp.float32),
        scratch_types=[
            pltpu.VMEM((chunk,), jnp.int32),
            pltpu.VMEM((chunk,), jnp.int32),
            pltpu.VMEM((chunk, d), jnp.float32),
            pltpu.VMEM((chunk, d), jnp.float32),
            pltpu.SemaphoreType.DMA,
            pltpu.SemaphoreType.DMA,
            pltpu.SemaphoreType.DMA,
            pltpu.SemaphoreType.DMA,
        ],
    )
    def gather_kernel(ids_hbm, table_hbm, out_hbm, idx_a, idx_b, rows_a,
                      rows_b, sem_ga, sem_gb, sem_wa, sem_wb):
        wid = lax.axis_index("s") * nc + lax.axis_index("c")
        base0 = wid * tok_per_w

        def out_at(tok):
            return out_hbm.at[pl.ds(tok, chunk)]

        pltpu.sync_copy(ids_hbm.at[pl.ds(base0, chunk)], idx_a)
        pltpu.async_copy(table_hbm.at[idx_a], rows_a, sem_ga)

        def body(g, carry):
            tok_a = base0 + (2 * g) * chunk
            tok_b = tok_a + chunk
            pltpu.make_async_copy(table_hbm.at[idx_a], rows_a, sem_ga).wait()

            @pl.when(g > 0)
            def _():
                pltpu.make_async_copy(rows_b, out_at(tok_b - 2 * chunk),
                                      sem_wb).wait()

            wr_a = pltpu.async_copy(rows_a, out_at(tok_a), sem_wa)
            pltpu.sync_copy(ids_hbm.at[pl.ds(tok_b, chunk)], idx_b)
            pltpu.async_copy(table_hbm.at[idx_b], rows_b, sem_gb)
            pltpu.make_async_copy(table_hbm.at[idx_b], rows_b, sem_gb).wait()
            wr_a.wait()
            pltpu.async_copy(rows_b, out_at(tok_b), sem_wb)

            @pl.when(g < n_pairs - 1)
            def _():
                tok_n = tok_b + chunk
                pltpu.sync_copy(ids_hbm.at[pl.ds(tok_n, chunk)], idx_a)
                pltpu.async_copy(table_hbm.at[idx_a], rows_a, sem_ga)

            return carry

        lax.fori_loop(0, n_pairs, body, 0)
        pltpu.make_async_copy(
            rows_b, out_at(base0 + tok_per_w - chunk), sem_wb).wait()

    return gather_kernel(ids_flat, word_table)


def _tc_epilogue(word_rows, type_ids, type_table, pos_emb, gamma, beta,
                 b_total, b_base, out_prev):
    b, s, d = word_rows.shape
    t = type_table.shape[0]

    def body(*refs):
        ids_ref, x_ref, ttab_ref, pos_ref, gamma_ref, beta_ref = refs[:6]
        out_ref, buf, sem = refs[-3], refs[-2], refs[-1]
        i = pl.program_id(0)
        slot = lax.rem(i, 2)

        def wr_copy(step, s_slot):
            return pltpu.make_async_copy(
                buf.at[s_slot], out_ref.at[pl.ds(b_base + step, 1)],
                sem.at[s_slot])

        @pl.when(i >= 2)
        def _():
            wr_copy(i - 2, slot).wait()

        x = x_ref[0]
        ids = ids_ref[0, 0]
        iota_t = lax.broadcasted_iota(jnp.int32, (1, t), 1)
        onehot = (ids[:, None] == iota_t).astype(jnp.float32)
        type_rows = jnp.dot(onehot, ttab_ref[...],
                            preferred_element_type=jnp.float32)
        y = x + type_rows + pos_ref[...]
        mean = jnp.mean(y, axis=-1, keepdims=True)
        c = y - mean
        var = jnp.mean(c * c, axis=-1, keepdims=True)
        norm = c * lax.rsqrt(var + 1e-12)
        res = (norm * gamma_ref[...] + beta_ref[...])[None]

        @pl.when(slot == 0)
        def _():
            buf[0] = res

        @pl.when(slot == 1)
        def _():
            buf[1] = res

        wr_copy(i, slot).start()

        @pl.when(i == b - 1)
        def _():
            @pl.when(jnp.int32(b) >= 2)
            def _():
                wr_copy(i - 1, 1 - slot).wait()
            wr_copy(i, slot).wait()

    in_specs = [
        pl.BlockSpec((1, 1, s), lambda i: (i, 0, 0)),
        pl.BlockSpec((1, s, d), lambda i: (i, 0, 0)),
        pl.BlockSpec((t, d), lambda i: (0, 0)),
        pl.BlockSpec((s, d), lambda i: (0, 0)),
        pl.BlockSpec((1, d), lambda i: (0, 0)),
        pl.BlockSpec((1, d), lambda i: (0, 0)),
    ]
    args = [type_ids.reshape(b, 1, s).astype(jnp.int32), word_rows,
            type_table, pos_emb, gamma.reshape(1, d), beta.reshape(1, d)]
    aliases = {}
    if out_prev is not None:
        in_specs.append(pl.BlockSpec(memory_space=pl.ANY))
        args.append(out_prev)
        aliases = {6: 0}
    return pl.pallas_call(
        body,
        grid=(b,),
        in_specs=in_specs,
        out_specs=pl.BlockSpec(memory_space=pl.ANY),
        out_shape=jax.ShapeDtypeStruct((b_total, s, d), jnp.float32),
        input_output_aliases=aliases,
        scratch_shapes=[
            pltpu.VMEM((2, 1, s, d), jnp.float32),
            pltpu.SemaphoreType.DMA((2,)),
        ],
    )(*args)


def kernel(input_word_ids, input_type_ids, word_table, type_table, pos_emb,
           gamma, beta):
    b, s = input_word_ids.shape
    v, d = word_table.shape
    sizes = [16] * 8
    assert sum(sizes) == b
    ids32 = input_word_ids.astype(jnp.int32)
    type32 = input_type_ids.astype(jnp.int32)
    starts = [sum(sizes[:k]) for k in range(len(sizes))]
    rows = [
        _sc_gather(ids32[b0:b0 + bs].reshape(-1), word_table,
                   bs * s, d).reshape(bs, s, d)
        for b0, bs in zip(starts, sizes)
    ]
    out = None
    for k, (b0, bs) in enumerate(zip(starts, sizes)):
        out = _tc_epilogue(rows[k], type32[b0:b0 + bs], type_table,
                           pos_emb, gamma, beta, b, b0, out)
    return out
